# baseline (speedup 1.0000x reference)
import functools

import jax
import jax.numpy as jnp
from jax import lax
from jax.experimental import pallas as pl
from jax.experimental.pallas import tpu as pltpu
from jax.experimental.pallas import tpu_sc as plsc

N = 10000
E = 320000
D_IN = 128
D_H = 256

CHUNK = 32
E_PAD = 327680
EROWS = E_PAD // CHUNK
NSUB = 16
NCORE = 2
N_PAD = 10240
STRIPE = N_PAD // NSUB

DH2 = D_H // 2
RING = 8
IB0 = 40
IB1 = 40

_mesh = plsc.VectorSubcoreMesh(core_axis_name="c", subcore_axis_name="s")


_ROWS0 = EROWS // (NCORE * NSUB)


def _ring_loop(table, src_v, dst_v, bufs, sems, acc, ib, extra=None):
    for k in range(RING):
        pltpu.async_copy(table.at[src_v.at[k]], bufs[k], sems[k])
    P = ib // RING

    def it(p, _):
        for k in range(RING):
            jk = RING * p + k
            pltpu.make_async_copy(table.at[src_v.at[jk]], bufs[k],
                                  sems[k]).wait()
            pltpu.async_copy(bufs[k], acc.at[dst_v.at[jk]], sems[k],
                             add=True)
            if extra is not None:
                ones_v, dacc, semd = extra
                pltpu.async_copy(ones_v, dacc.at[dst_v.at[jk]], semd,
                                 add=True)

        @pl.when(p + 1 < P)
        def _():
            for k in range(RING):
                jk = RING * p + k
                pltpu.make_async_copy(bufs[k], acc.at[dst_v.at[jk]],
                                      sems[k]).wait()
                pltpu.async_copy(table.at[src_v.at[jk + RING]], bufs[k],
                                 sems[k])
        return ()

    lax.fori_loop(0, P, it, (), unroll=False)
    for k in range(RING):
        pltpu.make_async_copy(bufs[k], acc.at[dst_v.at[k]], sems[k]).wait()


@functools.partial(
    pl.kernel,
    mesh=_mesh,
    out_type=[
        jax.ShapeDtypeStruct((NCORE, N_PAD, D_IN), jnp.float32),
        jax.ShapeDtypeStruct((NCORE, N_PAD), jnp.float32),
    ],
    scratch_types=[
        pltpu.VMEM((IB0, CHUNK), jnp.int32),
        pltpu.VMEM((IB0, CHUNK), jnp.int32),
        pltpu.VMEM((CHUNK, D_IN), jnp.float32),
        pltpu.VMEM((CHUNK, D_IN), jnp.float32),
        pltpu.VMEM((CHUNK, D_IN), jnp.float32),
        pltpu.VMEM((CHUNK, D_IN), jnp.float32),
        pltpu.VMEM((CHUNK, D_IN), jnp.float32),
        pltpu.VMEM((CHUNK, D_IN), jnp.float32),
        pltpu.VMEM((CHUNK, D_IN), jnp.float32),
        pltpu.VMEM((CHUNK, D_IN), jnp.float32),
        pltpu.VMEM((CHUNK,), jnp.float32),
        pltpu.VMEM_SHARED((N_PAD, D_IN), jnp.float32),
        pltpu.VMEM_SHARED((N_PAD,), jnp.float32),
        pltpu.SemaphoreType.DMA,
        pltpu.SemaphoreType.DMA,
        pltpu.SemaphoreType.DMA,
        pltpu.SemaphoreType.DMA,
        pltpu.SemaphoreType.DMA,
        pltpu.SemaphoreType.DMA,
        pltpu.SemaphoreType.DMA,
        pltpu.SemaphoreType.DMA,
        pltpu.SemaphoreType.DMA,
    ],
)
def _sc_agg0(x_hbm, src_hbm, dst_hbm, zero_hbm, zero1_hbm, out_hbm, deg_hbm,
             src_v, dst_v, r0, r1, r2, r3, r4, r5, r6, r7, ones_v, acc, dacc,
             s0, s1, s2, s3, s4, s5, s6, s7, semd):
    c = lax.axis_index("c")
    s = lax.axis_index("s")
    bufs = (r0, r1, r2, r3, r4, r5, r6, r7)
    sems = (s0, s1, s2, s3, s4, s5, s6, s7)
    pltpu.sync_copy(zero_hbm.at[pl.ds(s * STRIPE, STRIPE)],
                    acc.at[pl.ds(s * STRIPE, STRIPE)])
    pltpu.sync_copy(zero1_hbm.at[pl.ds(s * STRIPE, STRIPE)],
                    dacc.at[pl.ds(s * STRIPE, STRIPE)])
    base = (c * NSUB + s) * _ROWS0
    one = jnp.ones((16,), jnp.float32)
    for k in range(CHUNK // 16):
        ones_v[pl.ds(k * 16, 16)] = one
    plsc.subcore_barrier()

    def outer(b, _):
        pltpu.sync_copy(src_hbm.at[pl.ds(base + b * IB0, IB0)], src_v)
        pltpu.sync_copy(dst_hbm.at[pl.ds(base + b * IB0, IB0)], dst_v)
        _ring_loop(x_hbm, src_v, dst_v, bufs, sems, acc, IB0,
                   extra=(ones_v, dacc, semd))

        def draind(j, _):
            pltpu.make_async_copy(ones_v, dacc.at[dst_v.at[0]], semd).wait()
            return ()

        lax.fori_loop(0, IB0, draind, (), unroll=False)
        return ()

    lax.fori_loop(0, _ROWS0 // IB0, outer, (), unroll=False)
    plsc.subcore_barrier()
    pltpu.sync_copy(acc.at[pl.ds(s * STRIPE, STRIPE)],
                    out_hbm.at[c, pl.ds(s * STRIPE, STRIPE)])
    pltpu.sync_copy(dacc.at[pl.ds(s * STRIPE, STRIPE)],
                    deg_hbm.at[c, pl.ds(s * STRIPE, STRIPE)])


_ROWS1 = EROWS // NSUB


@functools.partial(
    pl.kernel,
    mesh=_mesh,
    out_type=jax.ShapeDtypeStruct((NCORE, N_PAD, DH2), jnp.float32),
    scratch_types=[
        pltpu.VMEM((IB1, CHUNK), jnp.int32),
        pltpu.VMEM((IB1, CHUNK), jnp.int32),
        pltpu.VMEM((CHUNK, DH2), jnp.float32),
        pltpu.VMEM((CHUNK, DH2), jnp.float32),
        pltpu.VMEM((CHUNK, DH2), jnp.float32),
        pltpu.VMEM((CHUNK, DH2), jnp.float32),
        pltpu.VMEM((CHUNK, DH2), jnp.float32),
        pltpu.VMEM((CHUNK, DH2), jnp.float32),
        pltpu.VMEM((CHUNK, DH2), jnp.float32),
        pltpu.VMEM((CHUNK, DH2), jnp.float32),
        pltpu.VMEM_SHARED((N_PAD, DH2), jnp.float32),
        pltpu.SemaphoreType.DMA,
        pltpu.SemaphoreType.DMA,
        pltpu.SemaphoreType.DMA,
        pltpu.SemaphoreType.DMA,
        pltpu.SemaphoreType.DMA,
        pltpu.SemaphoreType.DMA,
        pltpu.SemaphoreType.DMA,
        pltpu.SemaphoreType.DMA,
    ],
)
def _sc_agg_half(hl_hbm, hr_hbm, src_hbm, dst_hbm, zero_hbm, out_hbm,
                 src_v, dst_v, r0, r1, r2, r3, r4, r5, r6, r7, acc,
                 s0, s1, s2, s3, s4, s5, s6, s7):
    c = lax.axis_index("c")
    s = lax.axis_index("s")
    bufs = (r0, r1, r2, r3, r4, r5, r6, r7)
    sems = (s0, s1, s2, s3, s4, s5, s6, s7)
    pltpu.sync_copy(zero_hbm.at[pl.ds(s * STRIPE, STRIPE)],
                    acc.at[pl.ds(s * STRIPE, STRIPE)])
    base = s * _ROWS1
    plsc.subcore_barrier()

    def run(table):
        def outer(b, _):
            pltpu.sync_copy(src_hbm.at[pl.ds(base + b * IB1, IB1)], src_v)
            pltpu.sync_copy(dst_hbm.at[pl.ds(base + b * IB1, IB1)], dst_v)
            _ring_loop(table, src_v, dst_v, bufs, sems, acc, IB1)
            return ()

        lax.fori_loop(0, _ROWS1 // IB1, outer, (), unroll=False)

    @pl.when(c == 0)
    def _():
        run(hl_hbm)

    @pl.when(c == 1)
    def _():
        run(hr_hbm)

    plsc.subcore_barrier()
    pltpu.sync_copy(acc.at[pl.ds(s * STRIPE, STRIPE)],
                    out_hbm.at[c, pl.ds(s * STRIPE, STRIPE)])


_R = 1000


def _prelu(o, a):
    return jnp.where(o > 0.0, o, a * o)


def _dot_t(a, w):
    return lax.dot_general(a, w, (((1,), (1,)), ((), ())),
                           precision=lax.Precision.HIGHEST,
                           preferred_element_type=jnp.float32)


def _tc0_body(parts_ref, deg_ref, x_ref, w_ref, b_ref, a_ref,
              hl_ref, hr_ref, invd_ref):
    p = parts_ref[0] + parts_ref[1]
    invd = 1.0 / jnp.maximum(deg_ref[...], 1.0)
    cat = jnp.concatenate([p * invd, x_ref[...]], axis=1)
    out = _dot_t(cat, w_ref[...]) + b_ref[...]
    out = _prelu(out, a_ref[...])
    hl_ref[...] = out[:, :DH2]
    hr_ref[...] = out[:, DH2:]
    invd_ref[...] = invd


def _tc_mid_body(parts_ref, hl_in, hr_in, invd_ref, w_ref, b_ref, a_ref,
                 hl_ref, hr_ref):
    invd = invd_ref[...]
    cat = jnp.concatenate(
        [parts_ref[0] * invd, parts_ref[1] * invd, hl_in[...], hr_in[...]],
        axis=1)
    out = _prelu(_dot_t(cat, w_ref[...]) + b_ref[...], a_ref[...])
    hl_ref[...] = out[:, :DH2]
    hr_ref[...] = out[:, DH2:]


def _tc_last_body(parts_ref, hl_in, hr_in, invd_ref, w_ref, b_ref, a_ref,
                  out_ref):
    invd = invd_ref[...]
    cat = jnp.concatenate(
        [parts_ref[0] * invd, parts_ref[1] * invd, hl_in[...], hr_in[...]],
        axis=1)
    out_ref[...] = _prelu(_dot_t(cat, w_ref[...]) + b_ref[...], a_ref[...])


def _row_spec(d):
    return pl.BlockSpec((_R, d), lambda i: (i, 0))


def _whole_spec(shape):
    return pl.BlockSpec(shape, lambda i: tuple(0 for _ in shape))


def _tc0(parts, deg, x, wcat, b, alpha):
    return pl.pallas_call(
        _tc0_body,
        grid=(N // _R,),
        in_specs=[
            pl.BlockSpec((NCORE, _R, D_IN), lambda i: (0, i, 0)),
            _row_spec(1),
            _row_spec(D_IN),
            _whole_spec((D_H, 2 * D_IN)),
            _whole_spec((1, D_H)),
            _whole_spec((1, D_H)),
        ],
        out_specs=[_row_spec(DH2), _row_spec(DH2), _row_spec(1)],
        out_shape=[
            jax.ShapeDtypeStruct((N, DH2), jnp.float32),
            jax.ShapeDtypeStruct((N, DH2), jnp.float32),
            jax.ShapeDtypeStruct((N, 1), jnp.float32),
        ],
    )(parts, deg, x, wcat, b, alpha)


def _tc_mid(parts, hl, hr, invd, wcat, b, alpha):
    return pl.pallas_call(
        _tc_mid_body,
        grid=(N // _R,),
        in_specs=[
            pl.BlockSpec((NCORE, _R, DH2), lambda i: (0, i, 0)),
            _row_spec(DH2),
            _row_spec(DH2),
            _row_spec(1),
            _whole_spec((D_H, 2 * D_H)),
            _whole_spec((1, D_H)),
            _whole_spec((1, D_H)),
        ],
        out_specs=[_row_spec(DH2), _row_spec(DH2)],
        out_shape=[
            jax.ShapeDtypeStruct((N, DH2), jnp.float32),
            jax.ShapeDtypeStruct((N, DH2), jnp.float32),
        ],
    )(parts, hl, hr, invd, wcat, b, alpha)


def _tc_last(parts, hl, hr, invd, wcat, b, alpha):
    return pl.pallas_call(
        _tc_last_body,
        grid=(N // _R,),
        in_specs=[
            pl.BlockSpec((NCORE, _R, DH2), lambda i: (0, i, 0)),
            _row_spec(DH2),
            _row_spec(DH2),
            _row_spec(1),
            _whole_spec((D_H, 2 * D_H)),
            _whole_spec((1, D_H)),
            _whole_spec((1, D_H)),
        ],
        out_specs=_row_spec(D_H),
        out_shape=jax.ShapeDtypeStruct((N, D_H), jnp.float32),
    )(parts, hl, hr, invd, wcat, b, alpha)


def kernel(x, edge_index, W_l0, b_l0, W_r0, alpha0,
           W_l1, b_l1, W_r1, alpha1, W_l2, b_l2, W_r2, alpha2):
    npad = E_PAD - E
    pad_src = jnp.arange(npad, dtype=jnp.int32) % N
    pad_dst = N + jnp.arange(npad, dtype=jnp.int32) % (N_PAD - N)
    src = jnp.concatenate([edge_index[0], pad_src]).reshape(EROWS, CHUNK)
    dst = jnp.concatenate([edge_index[1], pad_dst]).reshape(EROWS, CHUNK)
    zh = jnp.zeros((N_PAD, DH2), jnp.float32)
    z1 = jnp.zeros((N_PAD,), jnp.float32)

    w0 = jnp.concatenate([W_l0, W_r0], axis=1)
    w1 = jnp.concatenate([W_l1, W_r1], axis=1)
    w2 = jnp.concatenate([W_l2, W_r2], axis=1)

    parts0, degp = _sc_agg0(x, src, dst, zh, z1)
    deg = (degp[0] + degp[1])[:N].reshape(N, 1)
    hl, hr, invd = _tc0(parts0, deg, x, w0,
                        b_l0.reshape(1, D_H), alpha0.reshape(1, D_H))

    parts1 = _sc_agg_half(hl, hr, src, dst, zh)
    hl, hr = _tc_mid(parts1, hl, hr, invd, w1,
                     b_l1.reshape(1, D_H), alpha1.reshape(1, D_H))

    parts2 = _sc_agg_half(hl, hr, src, dst, zh)
    return _tc_last(parts2, hl, hr, invd, w2,
                    b_l2.reshape(1, D_H), alpha2.reshape(1, D_H))


# speedup vs baseline: 1.0805x; 1.0805x over previous
import functools

import jax
import jax.numpy as jnp
from jax import lax
from jax.experimental import pallas as pl
from jax.experimental.pallas import tpu as pltpu
from jax.experimental.pallas import tpu_sc as plsc

N = 10000
E = 320000
D_IN = 128
D_H = 256

CHUNK = 64
E_PAD = 327680
EROWS = E_PAD // CHUNK
NSUB = 16
NCORE = 2
N_PAD = 10240
STRIPE = N_PAD // NSUB

DH2 = D_H // 2
RING = 4
IB0 = 32
IB1 = 64

_mesh = plsc.VectorSubcoreMesh(core_axis_name="c", subcore_axis_name="s")


_ROWS0 = EROWS // (NCORE * NSUB)


def _zero_fill(buf, nvec):
    zero16 = jnp.zeros((16,), jnp.float32)
    cols = buf.shape[-1] if len(buf.shape) == 2 else buf.shape[0]

    def body(i, _):
        if len(buf.shape) == 2:
            r = i // (buf.shape[1] // 16)
            k = lax.rem(i, buf.shape[1] // 16)
            buf[r, pl.ds(k * 16, 16)] = zero16
        else:
            buf[pl.ds(i * 16, 16)] = zero16
        return ()

    lax.fori_loop(0, nvec, body, (), unroll=False)


def _ring_loop(table, src_v, dst_v, bufs, sems, acc, ib, extra=None):
    for k in range(RING):
        pltpu.async_copy(table.at[src_v.at[k]], bufs[k], sems[k])
    P = ib // RING

    def it(p, _):
        for k in range(RING):
            jk = RING * p + k
            pltpu.make_async_copy(table.at[src_v.at[jk]], bufs[k],
                                  sems[k]).wait()
            pltpu.async_copy(bufs[k], acc.at[dst_v.at[jk]], sems[k],
                             add=True)
            if extra is not None:
                ones_v, dacc, semd = extra
                pltpu.async_copy(ones_v, dacc.at[dst_v.at[jk]], semd,
                                 add=True)

        @pl.when(p + 1 < P)
        def _():
            for k in range(RING):
                jk = RING * p + k
                pltpu.make_async_copy(bufs[k], acc.at[dst_v.at[jk]],
                                      sems[k]).wait()
                pltpu.async_copy(table.at[src_v.at[jk + RING]], bufs[k],
                                 sems[k])
        return ()

    lax.fori_loop(0, P, it, (), unroll=False)
    for k in range(RING):
        pltpu.make_async_copy(bufs[k], acc.at[dst_v.at[k]], sems[k]).wait()


@functools.partial(
    pl.kernel,
    mesh=_mesh,
    out_type=[
        jax.ShapeDtypeStruct((NCORE, N_PAD, D_IN), jnp.float32),
        jax.ShapeDtypeStruct((NCORE, N_PAD), jnp.float32),
    ],
    scratch_types=[
        pltpu.VMEM((IB0, CHUNK), jnp.int32),
        pltpu.VMEM((IB0, CHUNK), jnp.int32),
        pltpu.VMEM((CHUNK, D_IN), jnp.float32),
        pltpu.VMEM((CHUNK, D_IN), jnp.float32),
        pltpu.VMEM((CHUNK, D_IN), jnp.float32),
        pltpu.VMEM((CHUNK, D_IN), jnp.float32),
        pltpu.VMEM((CHUNK,), jnp.float32),
        pltpu.VMEM((STRIPE,), jnp.float32),
        pltpu.VMEM_SHARED((N_PAD, D_IN), jnp.float32),
        pltpu.VMEM_SHARED((N_PAD,), jnp.float32),
        pltpu.SemaphoreType.DMA,
        pltpu.SemaphoreType.DMA,
        pltpu.SemaphoreType.DMA,
        pltpu.SemaphoreType.DMA,
        pltpu.SemaphoreType.DMA,
    ],
)
def _sc_agg0(x_hbm, src_hbm, dst_hbm, out_hbm, deg_hbm,
             src_v, dst_v, r0, r1, r2, r3, ones_v, zbuf, acc, dacc,
             s0, s1, s2, s3, semd):
    c = lax.axis_index("c")
    s = lax.axis_index("s")
    bufs = (r0, r1, r2, r3)
    sems = (s0, s1, s2, s3)
    _zero_fill(r0, CHUNK * D_IN // 16)
    for q in range(STRIPE // CHUNK):
        pltpu.sync_copy(r0, acc.at[pl.ds(s * STRIPE + q * CHUNK, CHUNK)])
    _zero_fill(zbuf, STRIPE // 16)
    pltpu.sync_copy(zbuf, dacc.at[pl.ds(s * STRIPE, STRIPE)])
    base = (c * NSUB + s) * _ROWS0
    one = jnp.ones((16,), jnp.float32)
    for k in range(CHUNK // 16):
        ones_v[pl.ds(k * 16, 16)] = one
    plsc.subcore_barrier()

    def outer(b, _):
        pltpu.sync_copy(src_hbm.at[pl.ds(base + b * IB0, IB0)], src_v)
        pltpu.sync_copy(dst_hbm.at[pl.ds(base + b * IB0, IB0)], dst_v)
        _ring_loop(x_hbm, src_v, dst_v, bufs, sems, acc, IB0,
                   extra=(ones_v, dacc, semd))

        def draind(j, _):
            pltpu.make_async_copy(ones_v, dacc.at[dst_v.at[0]], semd).wait()
            return ()

        lax.fori_loop(0, IB0, draind, (), unroll=False)
        return ()

    lax.fori_loop(0, _ROWS0 // IB0, outer, (), unroll=False)
    plsc.subcore_barrier()
    pltpu.sync_copy(acc.at[pl.ds(s * STRIPE, STRIPE)],
                    out_hbm.at[c, pl.ds(s * STRIPE, STRIPE)])
    pltpu.sync_copy(dacc.at[pl.ds(s * STRIPE, STRIPE)],
                    deg_hbm.at[c, pl.ds(s * STRIPE, STRIPE)])


_ROWS1 = EROWS // NSUB


@functools.partial(
    pl.kernel,
    mesh=_mesh,
    out_type=jax.ShapeDtypeStruct((NCORE, N_PAD, DH2), jnp.float32),
    scratch_types=[
        pltpu.VMEM((IB1, CHUNK), jnp.int32),
        pltpu.VMEM((IB1, CHUNK), jnp.int32),
        pltpu.VMEM((CHUNK, DH2), jnp.float32),
        pltpu.VMEM((CHUNK, DH2), jnp.float32),
        pltpu.VMEM((CHUNK, DH2), jnp.float32),
        pltpu.VMEM((CHUNK, DH2), jnp.float32),
        pltpu.VMEM_SHARED((N_PAD, DH2), jnp.float32),
        pltpu.SemaphoreType.DMA,
        pltpu.SemaphoreType.DMA,
        pltpu.SemaphoreType.DMA,
        pltpu.SemaphoreType.DMA,
    ],
)
def _sc_agg_half(hl_hbm, hr_hbm, src_hbm, dst_hbm, out_hbm,
                 src_v, dst_v, r0, r1, r2, r3, acc,
                 s0, s1, s2, s3):
    c = lax.axis_index("c")
    s = lax.axis_index("s")
    bufs = (r0, r1, r2, r3)
    sems = (s0, s1, s2, s3)
    _zero_fill(r0, CHUNK * DH2 // 16)
    for q in range(STRIPE // CHUNK):
        pltpu.sync_copy(r0, acc.at[pl.ds(s * STRIPE + q * CHUNK, CHUNK)])
    base = s * _ROWS1
    plsc.subcore_barrier()

    def run(table):
        def outer(b, _):
            pltpu.sync_copy(src_hbm.at[pl.ds(base + b * IB1, IB1)], src_v)
            pltpu.sync_copy(dst_hbm.at[pl.ds(base + b * IB1, IB1)], dst_v)
            _ring_loop(table, src_v, dst_v, bufs, sems, acc, IB1)
            return ()

        lax.fori_loop(0, _ROWS1 // IB1, outer, (), unroll=False)

    @pl.when(c == 0)
    def _():
        run(hl_hbm)

    @pl.when(c == 1)
    def _():
        run(hr_hbm)

    plsc.subcore_barrier()
    pltpu.sync_copy(acc.at[pl.ds(s * STRIPE, STRIPE)],
                    out_hbm.at[c, pl.ds(s * STRIPE, STRIPE)])


_R = 1000


def _prelu(o, a):
    return jnp.where(o > 0.0, o, a * o)


def _dot_t(a, w):
    return lax.dot_general(a, w, (((1,), (1,)), ((), ())),
                           precision=lax.Precision.HIGHEST,
                           preferred_element_type=jnp.float32)


def _tc0_body(parts_ref, deg_ref, x_ref, w_ref, b_ref, a_ref,
              hl_ref, hr_ref, invd_ref):
    p = parts_ref[0] + parts_ref[1]
    invd = 1.0 / jnp.maximum(deg_ref[...], 1.0)
    cat = jnp.concatenate([p * invd, x_ref[...]], axis=1)
    out = _dot_t(cat, w_ref[...]) + b_ref[...]
    out = _prelu(out, a_ref[...])
    hl_ref[...] = out[:, :DH2]
    hr_ref[...] = out[:, DH2:]
    invd_ref[...] = invd


def _tc_mid_body(parts_ref, hl_in, hr_in, invd_ref, w_ref, b_ref, a_ref,
                 hl_ref, hr_ref):
    invd = invd_ref[...]
    cat = jnp.concatenate(
        [parts_ref[0] * invd, parts_ref[1] * invd, hl_in[...], hr_in[...]],
        axis=1)
    out = _prelu(_dot_t(cat, w_ref[...]) + b_ref[...], a_ref[...])
    hl_ref[...] = out[:, :DH2]
    hr_ref[...] = out[:, DH2:]


def _tc_last_body(parts_ref, hl_in, hr_in, invd_ref, w_ref, b_ref, a_ref,
                  out_ref):
    invd = invd_ref[...]
    cat = jnp.concatenate(
        [parts_ref[0] * invd, parts_ref[1] * invd, hl_in[...], hr_in[...]],
        axis=1)
    out_ref[...] = _prelu(_dot_t(cat, w_ref[...]) + b_ref[...], a_ref[...])


def _row_spec(d):
    return pl.BlockSpec((_R, d), lambda i: (i, 0))


def _whole_spec(shape):
    return pl.BlockSpec(shape, lambda i: tuple(0 for _ in shape))


def _tc0(parts, deg, x, wcat, b, alpha):
    return pl.pallas_call(
        _tc0_body,
        grid=(N // _R,),
        in_specs=[
            pl.BlockSpec((NCORE, _R, D_IN), lambda i: (0, i, 0)),
            _row_spec(1),
            _row_spec(D_IN),
            _whole_spec((D_H, 2 * D_IN)),
            _whole_spec((1, D_H)),
            _whole_spec((1, D_H)),
        ],
        out_specs=[_row_spec(DH2), _row_spec(DH2), _row_spec(1)],
        out_shape=[
            jax.ShapeDtypeStruct((N, DH2), jnp.float32),
            jax.ShapeDtypeStruct((N, DH2), jnp.float32),
            jax.ShapeDtypeStruct((N, 1), jnp.float32),
        ],
    )(parts, deg, x, wcat, b, alpha)


def _tc_mid(parts, hl, hr, invd, wcat, b, alpha):
    return pl.pallas_call(
        _tc_mid_body,
        grid=(N // _R,),
        in_specs=[
            pl.BlockSpec((NCORE, _R, DH2), lambda i: (0, i, 0)),
            _row_spec(DH2),
            _row_spec(DH2),
            _row_spec(1),
            _whole_spec((D_H, 2 * D_H)),
            _whole_spec((1, D_H)),
            _whole_spec((1, D_H)),
        ],
        out_specs=[_row_spec(DH2), _row_spec(DH2)],
        out_shape=[
            jax.ShapeDtypeStruct((N, DH2), jnp.float32),
            jax.ShapeDtypeStruct((N, DH2), jnp.float32),
        ],
    )(parts, hl, hr, invd, wcat, b, alpha)


def _tc_last(parts, hl, hr, invd, wcat, b, alpha):
    return pl.pallas_call(
        _tc_last_body,
        grid=(N // _R,),
        in_specs=[
            pl.BlockSpec((NCORE, _R, DH2), lambda i: (0, i, 0)),
            _row_spec(DH2),
            _row_spec(DH2),
            _row_spec(1),
            _whole_spec((D_H, 2 * D_H)),
            _whole_spec((1, D_H)),
            _whole_spec((1, D_H)),
        ],
        out_specs=_row_spec(D_H),
        out_shape=jax.ShapeDtypeStruct((N, D_H), jnp.float32),
    )(parts, hl, hr, invd, wcat, b, alpha)


def kernel(x, edge_index, W_l0, b_l0, W_r0, alpha0,
           W_l1, b_l1, W_r1, alpha1, W_l2, b_l2, W_r2, alpha2):
    npad = E_PAD - E
    pad_src = jnp.arange(npad, dtype=jnp.int32) % N
    pad_dst = N + jnp.arange(npad, dtype=jnp.int32) % (N_PAD - N)
    src = jnp.concatenate([edge_index[0], pad_src]).reshape(EROWS, CHUNK)
    dst = jnp.concatenate([edge_index[1], pad_dst]).reshape(EROWS, CHUNK)
    w0 = jnp.concatenate([W_l0, W_r0], axis=1)
    w1 = jnp.concatenate([W_l1, W_r1], axis=1)
    w2 = jnp.concatenate([W_l2, W_r2], axis=1)

    parts0, degp = _sc_agg0(x, src, dst)
    deg = (degp[0] + degp[1])[:N].reshape(N, 1)
    hl, hr, invd = _tc0(parts0, deg, x, w0,
                        b_l0.reshape(1, D_H), alpha0.reshape(1, D_H))

    parts1 = _sc_agg_half(hl, hr, src, dst)
    hl, hr = _tc_mid(parts1, hl, hr, invd, w1,
                     b_l1.reshape(1, D_H), alpha1.reshape(1, D_H))

    parts2 = _sc_agg_half(hl, hr, src, dst)
    return _tc_last(parts2, hl, hr, invd, w2,
                    b_l2.reshape(1, D_H), alpha2.reshape(1, D_H))
